# Initial kernel scaffold; baseline (speedup 1.0000x reference)
import jax
import jax.numpy as jnp
from jax.experimental import pallas as pl


def kernel(hidden_states, routing_weights, gate_up_proj, gate_up_proj_bias, down_proj, down_proj_bias, lora_A_gate_up, lora_B_gate_up, lora_A_down, lora_B_down, router_indices):
    raise NotImplementedError("write your pallas kernel here")


# speedup vs baseline: 3.4669x; 3.4669x over previous
import functools

import jax
import jax.numpy as jnp
from jax.experimental import pallas as pl
from jax.experimental.pallas import tpu as pltpu

SCALING = 32.0 / 16.0
ALPHA = 1.702
LIMIT = 7.0


def _expert_body(x_ref, wg_ref, wu_ref, bg_ref, bu_ref, agu_ref, bgg_ref,
                 bgu_ref, wd_ref, bd_ref, ad_ref, bdn_ref, coef_ref,
                 acc_ref, out_ref):
    x = x_ref[...]
    f32 = jnp.float32
    g = jnp.dot(x, wg_ref[...], preferred_element_type=f32) + bg_ref[...]
    u = jnp.dot(x, wu_ref[...], preferred_element_type=f32) + bu_ref[...]
    p = jnp.dot(x, agu_ref[...], preferred_element_type=f32)
    g = g + SCALING * jnp.dot(p, bgg_ref[...], preferred_element_type=f32)
    u = u + SCALING * jnp.dot(p, bgu_ref[...], preferred_element_type=f32)
    g = jnp.minimum(g, LIMIT)
    u = jnp.clip(u, -LIMIT, LIMIT)
    glu = g * jax.nn.sigmoid(g * ALPHA)
    gated = (u + 1.0) * glu
    y = jnp.dot(gated, wd_ref[...], preferred_element_type=f32) + bd_ref[...]
    q = jnp.dot(gated, ad_ref[...], preferred_element_type=f32)
    y = y + SCALING * jnp.dot(q, bdn_ref[...], preferred_element_type=f32)
    out_ref[...] = acc_ref[...] + coef_ref[...] * y


def kernel(hidden_states, routing_weights, gate_up_proj, gate_up_proj_bias,
           down_proj, down_proj_bias, lora_A_gate_up, lora_B_gate_up,
           lora_A_down, lora_B_down, router_indices):
    batch, seq, hd = hidden_states.shape
    num_experts, _, two_i = gate_up_proj.shape
    inner = two_i // 2
    rank = lora_A_gate_up.shape[-1]
    tokens = batch * seq
    tm = 256 if tokens % 256 == 0 else tokens

    x = hidden_states.reshape(tokens, hd)
    out = jnp.zeros_like(x)

    grid = (tokens // tm,)
    full = lambda shape: pl.BlockSpec(shape, lambda i: (0,) * len(shape))
    row = pl.BlockSpec((tm, hd), lambda i: (i, 0))

    call = pl.pallas_call(
        _expert_body,
        grid=grid,
        in_specs=[
            row,
            full((hd, inner)),
            full((hd, inner)),
            full((1, inner)),
            full((1, inner)),
            full((hd, rank)),
            full((rank, inner)),
            full((rank, inner)),
            full((inner, hd)),
            full((1, hd)),
            full((inner, rank)),
            full((rank, hd)),
            pl.BlockSpec((tm, 1), lambda i: (i, 0)),
            row,
        ],
        out_specs=row,
        out_shape=jax.ShapeDtypeStruct((tokens, hd), jnp.float32),
        input_output_aliases={13: 0},
    )

    for e in range(num_experts):
        wg = gate_up_proj[e, :, 0::2]
        wu = gate_up_proj[e, :, 1::2]
        bg = gate_up_proj_bias[e, 0::2][None, :]
        bu = gate_up_proj_bias[e, 1::2][None, :]
        bgg = lora_B_gate_up[e, :, 0::2]
        bgu = lora_B_gate_up[e, :, 1::2]
        cnt = (router_indices == e).sum(axis=1).astype(jnp.float32)
        coef = (cnt * routing_weights[:, e])[:, None]
        out = call(x, wg, wu, bg, bu, lora_A_gate_up[e], bgg, bgu,
                   down_proj[e], down_proj_bias[e][None, :],
                   lora_A_down[e], lora_B_down[e], coef, out)
    return out.reshape(batch, seq, hd)


# speedup vs baseline: 6.4222x; 1.8524x over previous
import functools

import jax
import jax.numpy as jnp
from jax import lax
from jax.experimental import pallas as pl
from jax.experimental.pallas import tpu as pltpu
from jax.experimental.pallas import tpu_sc as plsc

SCALING = 32.0 / 16.0
ALPHA = 1.702
LIMIT = 7.0

TM = 256
SC_CHUNK = 64


def _sc_gather_rows(table, idx):
    n_rows = idx.shape[0]
    width = table.shape[1]
    info = plsc.get_sparse_core_info()
    nc, ns = info.num_cores, info.num_subcores
    nw = nc * ns
    rows_per_w = n_rows // nw
    assert rows_per_w * nw == n_rows and rows_per_w % SC_CHUNK == 0

    mesh = plsc.VectorSubcoreMesh(core_axis_name="c", subcore_axis_name="s")

    @functools.partial(
        pl.kernel, mesh=mesh,
        out_type=jax.ShapeDtypeStruct((n_rows, width), jnp.float32),
        scratch_types=[
            pltpu.VMEM((SC_CHUNK,), jnp.int32),
            pltpu.VMEM((SC_CHUNK, width), jnp.float32),
            pltpu.SemaphoreType.DMA,
        ],
    )
    def gather_k(idx_hbm, table_hbm, out_hbm, idx_v, rows_v, sem):
        wid = lax.axis_index("s") * nc + lax.axis_index("c")
        base = wid * rows_per_w
        for c in range(rows_per_w // SC_CHUNK):
            off = base + c * SC_CHUNK
            pltpu.sync_copy(idx_hbm.at[pl.ds(off, SC_CHUNK)], idx_v)
            pltpu.async_copy(table_hbm.at[idx_v], rows_v, sem).wait()
            pltpu.sync_copy(rows_v, out_hbm.at[pl.ds(off, SC_CHUNK)])

    return gather_k(idx, table)


def _grouped_body(te_ref, x_ref, wg_ref, wu_ref, bg_ref, bu_ref, agu_ref,
                  bgg_ref, bgu_ref, wd_ref, bd_ref, ad_ref, bdn_ref, y_ref,
                  *, num_experts):
    i = pl.program_id(0)
    f32 = jnp.float32
    bf16 = jnp.bfloat16

    @pl.when(te_ref[i] < num_experts)
    def _():
        x = x_ref[...].astype(bf16)
        g = jnp.dot(x, wg_ref[0], preferred_element_type=f32) + bg_ref[0]
        u = jnp.dot(x, wu_ref[0], preferred_element_type=f32) + bu_ref[0]
        p = jnp.dot(x, agu_ref[0], preferred_element_type=f32).astype(bf16)
        g = g + SCALING * jnp.dot(p, bgg_ref[0], preferred_element_type=f32)
        u = u + SCALING * jnp.dot(p, bgu_ref[0], preferred_element_type=f32)
        g = jnp.minimum(g, LIMIT)
        u = jnp.clip(u, -LIMIT, LIMIT)
        glu = g * jax.nn.sigmoid(g * ALPHA)
        gated = ((u + 1.0) * glu).astype(bf16)
        y = jnp.dot(gated, wd_ref[0], preferred_element_type=f32) + bd_ref[0]
        q = jnp.dot(gated, ad_ref[0], preferred_element_type=f32).astype(bf16)
        y = y + SCALING * jnp.dot(q, bdn_ref[0], preferred_element_type=f32)
        y_ref[...] = y


def _combine_body(z_ref, w0_ref, w1_ref, o_ref, *, width):
    o_ref[...] = (w0_ref[...] * z_ref[:, :width]
                  + w1_ref[...] * z_ref[:, width:])


def kernel(hidden_states, routing_weights, gate_up_proj, gate_up_proj_bias,
           down_proj, down_proj_bias, lora_A_gate_up, lora_B_gate_up,
           lora_A_down, lora_B_down, router_indices):
    batch, seq, hd = hidden_states.shape
    num_experts, _, two_i = gate_up_proj.shape
    inner = two_i // 2
    rank = lora_A_gate_up.shape[-1]
    tokens = batch * seq
    topk = router_indices.shape[1]
    pairs = tokens * topk
    ntiles = pairs // TM + num_experts
    cap = ntiles * TM

    x = hidden_states.reshape(tokens, hd)
    f32 = jnp.float32
    bf16 = jnp.bfloat16

    e_pair = router_indices.reshape(-1).astype(jnp.int32)
    order = jnp.argsort(e_pair, stable=True).astype(jnp.int32)
    sorted_e = e_pair[order]
    eids = jnp.arange(num_experts, dtype=jnp.int32)
    n_e = jnp.sum(e_pair[None, :] == eids[:, None], axis=1).astype(jnp.int32)
    start_e = jnp.concatenate([jnp.zeros((1,), jnp.int32), jnp.cumsum(n_e)[:-1]])
    ntiles_e = (n_e + TM - 1) // TM
    cumtiles = jnp.cumsum(ntiles_e)
    padded_start_e = TM * jnp.concatenate(
        [jnp.zeros((1,), jnp.int32), cumtiles[:-1]])
    rank_in_group = jnp.arange(pairs, dtype=jnp.int32) - start_e[sorted_e]
    dst = padded_start_e[sorted_e] + rank_in_group
    row_token = jnp.zeros((cap,), jnp.int32).at[dst].set(order // topk)
    pair_pos = jnp.zeros((pairs,), jnp.int32).at[order].set(dst)
    tile_expert = jnp.searchsorted(
        cumtiles, jnp.arange(ntiles, dtype=jnp.int32), side="right"
    ).astype(jnp.int32)
    w_pair = routing_weights[
        jnp.arange(pairs, dtype=jnp.int32) // topk, e_pair].reshape(tokens, topk)

    wg = gate_up_proj[:, :, 0::2].astype(bf16)
    wu = gate_up_proj[:, :, 1::2].astype(bf16)
    bg = gate_up_proj_bias[:, None, 0::2]
    bu = gate_up_proj_bias[:, None, 1::2]
    agu = lora_A_gate_up.astype(bf16)
    bgg = lora_B_gate_up[:, :, 0::2].astype(bf16)
    bgu = lora_B_gate_up[:, :, 1::2].astype(bf16)
    wd = down_proj.astype(bf16)
    bd = down_proj_bias[:, None, :]
    ad = lora_A_down.astype(bf16)
    bdn = lora_B_down.astype(bf16)

    x_pad = _sc_gather_rows(x, row_token)

    def emap(e3):
        return lambda i, te: (jnp.minimum(te[i], num_experts - 1),) + (0,) * e3

    grid_spec = pltpu.PrefetchScalarGridSpec(
        num_scalar_prefetch=1,
        grid=(ntiles,),
        in_specs=[
            pl.BlockSpec((TM, hd), lambda i, te: (i, 0)),
            pl.BlockSpec((1, hd, inner), emap(2)),
            pl.BlockSpec((1, hd, inner), emap(2)),
            pl.BlockSpec((1, 1, inner), emap(2)),
            pl.BlockSpec((1, 1, inner), emap(2)),
            pl.BlockSpec((1, hd, rank), emap(2)),
            pl.BlockSpec((1, rank, inner), emap(2)),
            pl.BlockSpec((1, rank, inner), emap(2)),
            pl.BlockSpec((1, inner, hd), emap(2)),
            pl.BlockSpec((1, 1, hd), emap(2)),
            pl.BlockSpec((1, inner, rank), emap(2)),
            pl.BlockSpec((1, rank, hd), emap(2)),
        ],
        out_specs=pl.BlockSpec((TM, hd), lambda i, te: (i, 0)),
    )
    y_pad = pl.pallas_call(
        functools.partial(_grouped_body, num_experts=num_experts),
        grid_spec=grid_spec,
        out_shape=jax.ShapeDtypeStruct((cap, hd), f32),
    )(tile_expert, x_pad, wg, wu, bg, bu, agu, bgg, bgu, wd, bd, ad, bdn)

    z = _sc_gather_rows(y_pad, pair_pos)

    z2 = z.reshape(tokens, topk * hd)
    tm2 = min(512, tokens)
    out = pl.pallas_call(
        functools.partial(_combine_body, width=hd),
        grid=(tokens // tm2,),
        in_specs=[
            pl.BlockSpec((tm2, topk * hd), lambda i: (i, 0)),
            pl.BlockSpec((tm2, 1), lambda i: (i, 0)),
            pl.BlockSpec((tm2, 1), lambda i: (i, 0)),
        ],
        out_specs=pl.BlockSpec((tm2, hd), lambda i: (i, 0)),
        out_shape=jax.ShapeDtypeStruct((tokens, hd), f32),
    )(z2, w_pair[:, 0:1], w_pair[:, 1:2])

    return out.reshape(batch, seq, hd)


# speedup vs baseline: 21.3150x; 3.3190x over previous
import functools

import jax
import jax.numpy as jnp
from jax import lax
from jax.experimental import pallas as pl
from jax.experimental.pallas import tpu as pltpu
from jax.experimental.pallas import tpu_sc as plsc

SCALING = 32.0 / 16.0
ALPHA = 1.702
LIMIT = 7.0

TM = 128
SC_CHUNK = 64


def _sc_gather_rows(table, idx):
    n_rows = idx.shape[0]
    width = table.shape[1]
    info = plsc.get_sparse_core_info()
    nc, ns = info.num_cores, info.num_subcores
    nw = nc * ns
    rows_per_w = n_rows // nw
    assert rows_per_w * nw == n_rows
    chunk = next(c for c in (64, 48, 32, 16, 8) if rows_per_w % c == 0)

    mesh = plsc.VectorSubcoreMesh(core_axis_name="c", subcore_axis_name="s")

    @functools.partial(
        pl.kernel, mesh=mesh,
        out_type=jax.ShapeDtypeStruct((n_rows, width), jnp.float32),
        scratch_types=[
            pltpu.VMEM((chunk,), jnp.int32),
            pltpu.VMEM((chunk, width), jnp.float32),
            pltpu.SemaphoreType.DMA,
        ],
    )
    def gather_k(idx_hbm, table_hbm, out_hbm, idx_v, rows_v, sem):
        wid = lax.axis_index("s") * nc + lax.axis_index("c")
        base = wid * rows_per_w
        for c in range(rows_per_w // chunk):
            off = base + c * chunk
            pltpu.sync_copy(idx_hbm.at[pl.ds(off, chunk)], idx_v)
            pltpu.async_copy(table_hbm.at[idx_v], rows_v, sem).wait()
            pltpu.sync_copy(rows_v, out_hbm.at[pl.ds(off, chunk)])

    return gather_k(idx, table)


def _grouped_body(te_ref, x_ref, wgu_ref, bgu_b_ref, agu_ref, bgu_l_ref,
                  wd_ref, bd_ref, ad_ref, bdn_ref, y_ref, *, num_experts):
    i = pl.program_id(0)
    f32 = jnp.float32
    bf16 = jnp.bfloat16

    @pl.when(te_ref[i] < num_experts)
    def _():
        x = x_ref[...].astype(bf16)
        gu = jnp.dot(x, wgu_ref[0], preferred_element_type=f32) + bgu_b_ref[0]
        p = jnp.dot(x, agu_ref[0], preferred_element_type=f32).astype(bf16)
        gu = gu + SCALING * jnp.dot(p, bgu_l_ref[0], preferred_element_type=f32)
        g = jnp.minimum(gu, LIMIT)
        glu = g * jax.nn.sigmoid(g * ALPHA)
        u1 = jnp.clip(gu, -LIMIT, LIMIT) + 1.0
        u_shift = jnp.concatenate([u1[:, 1:], u1[:, :1]], axis=1)
        gated = (u_shift * glu).astype(bf16)
        y = jnp.dot(gated, wd_ref[0], preferred_element_type=f32) + bd_ref[0]
        q = jnp.dot(gated, ad_ref[0], preferred_element_type=f32).astype(bf16)
        y = y + SCALING * jnp.dot(q, bdn_ref[0], preferred_element_type=f32)
        y_ref[...] = y


def _combine_body(z_ref, w0_ref, w1_ref, o_ref, *, width):
    o_ref[...] = (w0_ref[...] * z_ref[:, :width]
                  + w1_ref[...] * z_ref[:, width:])


def kernel(hidden_states, routing_weights, gate_up_proj, gate_up_proj_bias,
           down_proj, down_proj_bias, lora_A_gate_up, lora_B_gate_up,
           lora_A_down, lora_B_down, router_indices):
    batch, seq, hd = hidden_states.shape
    num_experts, _, two_i = gate_up_proj.shape
    inner = two_i // 2
    rank = lora_A_gate_up.shape[-1]
    tokens = batch * seq
    topk = router_indices.shape[1]
    pairs = tokens * topk
    ntiles = pairs // TM + num_experts
    cap = ntiles * TM

    x = hidden_states.reshape(tokens, hd)
    f32 = jnp.float32
    bf16 = jnp.bfloat16

    e_pair = router_indices.reshape(-1).astype(jnp.int32)
    order = jnp.argsort(e_pair, stable=True).astype(jnp.int32)
    sorted_e = e_pair[order]
    eids = jnp.arange(num_experts, dtype=jnp.int32)
    n_e = jnp.sum(e_pair[None, :] == eids[:, None], axis=1).astype(jnp.int32)
    start_e = jnp.concatenate([jnp.zeros((1,), jnp.int32), jnp.cumsum(n_e)[:-1]])
    ntiles_e = (n_e + TM - 1) // TM
    cumtiles = jnp.cumsum(ntiles_e)
    padded_start_e = TM * jnp.concatenate(
        [jnp.zeros((1,), jnp.int32), cumtiles[:-1]])
    rank_in_group = jnp.arange(pairs, dtype=jnp.int32) - start_e[sorted_e]
    dst = padded_start_e[sorted_e] + rank_in_group
    row_token = jnp.zeros((cap,), jnp.int32).at[dst].set(order // topk)
    pair_pos = jnp.zeros((pairs,), jnp.int32).at[order].set(dst)
    tile_expert = jnp.searchsorted(
        cumtiles, jnp.arange(ntiles, dtype=jnp.int32), side="right"
    ).astype(jnp.int32)
    w_pair = routing_weights[
        jnp.arange(pairs, dtype=jnp.int32) // topk, e_pair].reshape(tokens, topk)

    wgu = gate_up_proj.astype(bf16)
    bgu_b = gate_up_proj_bias[:, None, :]
    agu = lora_A_gate_up.astype(bf16)
    bgu_l = lora_B_gate_up.astype(bf16)
    wd = jnp.stack([down_proj.astype(bf16),
                    jnp.zeros_like(down_proj, bf16)], axis=2
                   ).reshape(num_experts, two_i, hd)
    bd = down_proj_bias[:, None, :]
    ad = jnp.stack([lora_A_down.astype(bf16),
                    jnp.zeros_like(lora_A_down, bf16)], axis=2
                   ).reshape(num_experts, two_i, rank)
    bdn = lora_B_down.astype(bf16)

    x_pad = _sc_gather_rows(x, row_token)

    def emap(e3):
        return lambda i, te: (jnp.minimum(te[i], num_experts - 1),) + (0,) * e3

    grid_spec = pltpu.PrefetchScalarGridSpec(
        num_scalar_prefetch=1,
        grid=(ntiles,),
        in_specs=[
            pl.BlockSpec((TM, hd), lambda i, te: (i, 0)),
            pl.BlockSpec((1, hd, two_i), emap(2)),
            pl.BlockSpec((1, 1, two_i), emap(2)),
            pl.BlockSpec((1, hd, rank), emap(2)),
            pl.BlockSpec((1, rank, two_i), emap(2)),
            pl.BlockSpec((1, two_i, hd), emap(2)),
            pl.BlockSpec((1, 1, hd), emap(2)),
            pl.BlockSpec((1, two_i, rank), emap(2)),
            pl.BlockSpec((1, rank, hd), emap(2)),
        ],
        out_specs=pl.BlockSpec((TM, hd), lambda i, te: (i, 0)),
    )
    y_pad = pl.pallas_call(
        functools.partial(_grouped_body, num_experts=num_experts),
        grid_spec=grid_spec,
        out_shape=jax.ShapeDtypeStruct((cap, hd), f32),
    )(tile_expert, x_pad, wgu, bgu_b, agu, bgu_l, wd, bd, ad, bdn)

    z = _sc_gather_rows(y_pad, pair_pos)

    z2 = z.reshape(tokens, topk * hd)
    tm2 = min(512, tokens)
    out = pl.pallas_call(
        functools.partial(_combine_body, width=hd),
        grid=(tokens // tm2,),
        in_specs=[
            pl.BlockSpec((tm2, topk * hd), lambda i: (i, 0)),
            pl.BlockSpec((tm2, 1), lambda i: (i, 0)),
            pl.BlockSpec((tm2, 1), lambda i: (i, 0)),
        ],
        out_specs=pl.BlockSpec((tm2, hd), lambda i: (i, 0)),
        out_shape=jax.ShapeDtypeStruct((tokens, hd), f32),
    )(z2, w_pair[:, 0:1], w_pair[:, 1:2])

    return out.reshape(batch, seq, hd)


# speedup vs baseline: 25.5063x; 1.1966x over previous
import functools

import jax
import jax.numpy as jnp
from jax import lax
from jax.experimental import pallas as pl
from jax.experimental.pallas import tpu as pltpu
from jax.experimental.pallas import tpu_sc as plsc

SCALING = 32.0 / 16.0
ALPHA = 1.702
LIMIT = 7.0

TM = 128
SC_CHUNK = 64


def _sc_gather_rows(table, idx):
    n_rows = idx.shape[0]
    width = table.shape[1]
    info = plsc.get_sparse_core_info()
    nc, ns = info.num_cores, info.num_subcores
    nw = nc * ns
    rows_per_w = n_rows // nw
    assert rows_per_w * nw == n_rows
    chunk = next(c for c in (64, 48, 32, 16, 8) if rows_per_w % c == 0)

    mesh = plsc.VectorSubcoreMesh(core_axis_name="c", subcore_axis_name="s")

    @functools.partial(
        pl.kernel, mesh=mesh,
        out_type=jax.ShapeDtypeStruct((n_rows, width), jnp.float32),
        scratch_types=[
            pltpu.VMEM((chunk,), jnp.int32),
            pltpu.VMEM((chunk, width), jnp.float32),
            pltpu.SemaphoreType.DMA,
        ],
    )
    def gather_k(idx_hbm, table_hbm, out_hbm, idx_v, rows_v, sem):
        wid = lax.axis_index("s") * nc + lax.axis_index("c")
        base = wid * rows_per_w
        for c in range(rows_per_w // chunk):
            off = base + c * chunk
            pltpu.sync_copy(idx_hbm.at[pl.ds(off, chunk)], idx_v)
            pltpu.async_copy(table_hbm.at[idx_v], rows_v, sem).wait()
            pltpu.sync_copy(rows_v, out_hbm.at[pl.ds(off, chunk)])

    return gather_k(idx, table)


def _grouped_body(te_ref, x_ref, wgu_ref, bgu_b_ref, agu_ref, bgu_l_ref,
                  wd_ref, bd_ref, ad_ref, bdn_ref, y_ref, *, num_experts):
    i = pl.program_id(0)
    f32 = jnp.float32
    bf16 = jnp.bfloat16

    @pl.when(te_ref[i] < num_experts)
    def _():
        x = x_ref[...].astype(bf16)
        gu = jnp.dot(x, wgu_ref[0], preferred_element_type=f32) + bgu_b_ref[0]
        p = jnp.dot(x, agu_ref[0], preferred_element_type=f32).astype(bf16)
        gu = gu + SCALING * jnp.dot(p, bgu_l_ref[0], preferred_element_type=f32)
        g = jnp.minimum(gu, LIMIT)
        glu = g * jax.nn.sigmoid(g * ALPHA)
        u1 = jnp.clip(gu, -LIMIT, LIMIT) + 1.0
        u_shift = jnp.concatenate([u1[:, 1:], u1[:, :1]], axis=1)
        gated = (u_shift * glu).astype(bf16)
        y = jnp.dot(gated, wd_ref[0], preferred_element_type=f32) + bd_ref[0]
        q = jnp.dot(gated, ad_ref[0], preferred_element_type=f32).astype(bf16)
        y = y + SCALING * jnp.dot(q, bdn_ref[0], preferred_element_type=f32)
        y_ref[...] = y


def _expand_body(w_ref, o_ref, *, width):
    rows, owidth = o_ref.shape[1], o_ref.shape[2]
    o_ref[0, :, :width] = w_ref[0].astype(jnp.bfloat16)
    o_ref[0, :, width:] = jnp.zeros((rows, owidth - width), jnp.bfloat16)


def _combine_body(z_ref, w0_ref, w1_ref, o_ref, *, width):
    o_ref[...] = (w0_ref[...] * z_ref[:, :width]
                  + w1_ref[...] * z_ref[:, width:])


def kernel(hidden_states, routing_weights, gate_up_proj, gate_up_proj_bias,
           down_proj, down_proj_bias, lora_A_gate_up, lora_B_gate_up,
           lora_A_down, lora_B_down, router_indices):
    batch, seq, hd = hidden_states.shape
    num_experts, _, two_i = gate_up_proj.shape
    inner = two_i // 2
    rank = lora_A_gate_up.shape[-1]
    tokens = batch * seq
    topk = router_indices.shape[1]
    pairs = tokens * topk
    ntiles = pairs // TM + num_experts
    cap = ntiles * TM

    x = hidden_states.reshape(tokens, hd)
    f32 = jnp.float32
    bf16 = jnp.bfloat16

    e_pair = router_indices.reshape(-1).astype(jnp.int32)
    order = jnp.argsort(e_pair, stable=True).astype(jnp.int32)
    sorted_e = e_pair[order]
    eids = jnp.arange(num_experts, dtype=jnp.int32)
    n_e = jnp.sum(e_pair[None, :] == eids[:, None], axis=1).astype(jnp.int32)
    start_e = jnp.concatenate([jnp.zeros((1,), jnp.int32), jnp.cumsum(n_e)[:-1]])
    ntiles_e = (n_e + TM - 1) // TM
    cumtiles = jnp.cumsum(ntiles_e)
    padded_start_e = TM * jnp.concatenate(
        [jnp.zeros((1,), jnp.int32), cumtiles[:-1]])
    rank_in_group = jnp.arange(pairs, dtype=jnp.int32) - start_e[sorted_e]
    dst = padded_start_e[sorted_e] + rank_in_group
    row_token = jnp.zeros((cap,), jnp.int32).at[dst].set(order // topk)
    pair_pos = jnp.zeros((pairs,), jnp.int32).at[order].set(dst)
    tile_expert = jnp.searchsorted(
        cumtiles, jnp.arange(ntiles, dtype=jnp.int32), side="right"
    ).astype(jnp.int32)
    w_pair = routing_weights[
        jnp.arange(pairs, dtype=jnp.int32) // topk, e_pair].reshape(tokens, topk)

    wgu = gate_up_proj.astype(bf16)
    bgu_b = gate_up_proj_bias[:, None, :]
    agu = lora_A_gate_up.astype(bf16)
    bgu_l = lora_B_gate_up.astype(bf16)
    rt = 480 if inner % 480 == 0 else inner
    wd = pl.pallas_call(
        functools.partial(_expand_body, width=hd),
        grid=(num_experts, inner // rt),
        in_specs=[pl.BlockSpec((1, rt, hd), lambda e, r: (e, r, 0))],
        out_specs=pl.BlockSpec((1, rt, 2 * hd), lambda e, r: (e, r, 0)),
        out_shape=jax.ShapeDtypeStruct((num_experts, inner, 2 * hd), bf16),
    )(down_proj).reshape(num_experts, two_i, hd)
    bd = down_proj_bias[:, None, :]
    ad = pl.pallas_call(
        functools.partial(_expand_body, width=rank),
        grid=(num_experts, 1),
        in_specs=[pl.BlockSpec((1, inner, rank), lambda e, r: (e, r, 0))],
        out_specs=pl.BlockSpec((1, inner, 2 * rank), lambda e, r: (e, r, 0)),
        out_shape=jax.ShapeDtypeStruct((num_experts, inner, 2 * rank), bf16),
    )(lora_A_down).reshape(num_experts, two_i, rank)
    bdn = lora_B_down.astype(bf16)

    x_pad = _sc_gather_rows(x, row_token)

    def emap(e3):
        return lambda i, te: (jnp.minimum(te[i], num_experts - 1),) + (0,) * e3

    grid_spec = pltpu.PrefetchScalarGridSpec(
        num_scalar_prefetch=1,
        grid=(ntiles,),
        in_specs=[
            pl.BlockSpec((TM, hd), lambda i, te: (i, 0)),
            pl.BlockSpec((1, hd, two_i), emap(2)),
            pl.BlockSpec((1, 1, two_i), emap(2)),
            pl.BlockSpec((1, hd, rank), emap(2)),
            pl.BlockSpec((1, rank, two_i), emap(2)),
            pl.BlockSpec((1, two_i, hd), emap(2)),
            pl.BlockSpec((1, 1, hd), emap(2)),
            pl.BlockSpec((1, two_i, rank), emap(2)),
            pl.BlockSpec((1, rank, hd), emap(2)),
        ],
        out_specs=pl.BlockSpec((TM, hd), lambda i, te: (i, 0)),
    )
    y_pad = pl.pallas_call(
        functools.partial(_grouped_body, num_experts=num_experts),
        grid_spec=grid_spec,
        out_shape=jax.ShapeDtypeStruct((cap, hd), f32),
    )(tile_expert, x_pad, wgu, bgu_b, agu, bgu_l, wd, bd, ad, bdn)

    z = _sc_gather_rows(y_pad, pair_pos)

    z2 = z.reshape(tokens, topk * hd)
    tm2 = min(512, tokens)
    out = pl.pallas_call(
        functools.partial(_combine_body, width=hd),
        grid=(tokens // tm2,),
        in_specs=[
            pl.BlockSpec((tm2, topk * hd), lambda i: (i, 0)),
            pl.BlockSpec((tm2, 1), lambda i: (i, 0)),
            pl.BlockSpec((tm2, 1), lambda i: (i, 0)),
        ],
        out_specs=pl.BlockSpec((tm2, hd), lambda i: (i, 0)),
        out_shape=jax.ShapeDtypeStruct((tokens, hd), f32),
    )(z2, w_pair[:, 0:1], w_pair[:, 1:2])

    return out.reshape(batch, seq, hd)


# speedup vs baseline: 26.3973x; 1.0349x over previous
import functools

import jax
import jax.numpy as jnp
from jax import lax
from jax.experimental import pallas as pl
from jax.experimental.pallas import tpu as pltpu
from jax.experimental.pallas import tpu_sc as plsc

SCALING = 32.0 / 16.0
ALPHA = 1.702
LIMIT = 7.0

TM = 128
SC_CHUNK = 64


def _sc_gather_rows(table, idx):
    n_rows = idx.shape[0]
    width = table.shape[1]
    info = plsc.get_sparse_core_info()
    nc, ns = info.num_cores, info.num_subcores
    nw = nc * ns
    rows_per_w = n_rows // nw
    assert rows_per_w * nw == n_rows
    chunk = next(c for c in (64, 48, 32, 16, 8) if rows_per_w % c == 0)

    mesh = plsc.VectorSubcoreMesh(core_axis_name="c", subcore_axis_name="s")

    @functools.partial(
        pl.kernel, mesh=mesh,
        out_type=jax.ShapeDtypeStruct((n_rows, width), jnp.float32),
        scratch_types=[
            pltpu.VMEM((chunk,), jnp.int32),
            pltpu.VMEM((chunk, width), jnp.float32),
            pltpu.SemaphoreType.DMA,
        ],
    )
    def gather_k(idx_hbm, table_hbm, out_hbm, idx_v, rows_v, sem):
        wid = lax.axis_index("s") * nc + lax.axis_index("c")
        base = wid * rows_per_w
        for c in range(rows_per_w // chunk):
            off = base + c * chunk
            pltpu.sync_copy(idx_hbm.at[pl.ds(off, chunk)], idx_v)
            pltpu.async_copy(table_hbm.at[idx_v], rows_v, sem).wait()
            pltpu.sync_copy(rows_v, out_hbm.at[pl.ds(off, chunk)])

    return gather_k(idx, table)


def _grouped_body(te_ref, x_ref, wgu_ref, bgu_b_ref, agu_ref, bgu_l_ref,
                  wd_ref, bd_ref, ad_ref, bdn_ref, y_ref, *, num_experts):
    i = pl.program_id(0)
    f32 = jnp.float32
    bf16 = jnp.bfloat16

    @pl.when(te_ref[i] < num_experts)
    def _():
        x = x_ref[...].astype(bf16)
        gu = jnp.dot(x, wgu_ref[0], preferred_element_type=f32) + bgu_b_ref[0]
        p = jnp.dot(x, agu_ref[0], preferred_element_type=f32).astype(bf16)
        gu = gu + SCALING * jnp.dot(p, bgu_l_ref[0], preferred_element_type=f32)
        g = jnp.minimum(gu, LIMIT)
        glu = g * jax.nn.sigmoid(g * ALPHA)
        u1 = jnp.clip(gu, -LIMIT, LIMIT) + 1.0
        u_shift = jnp.concatenate([u1[:, 1:], u1[:, :1]], axis=1)
        gated = (u_shift * glu).astype(bf16)
        y = jnp.dot(gated, wd_ref[0], preferred_element_type=f32) + bd_ref[0]
        q = jnp.dot(gated, ad_ref[0], preferred_element_type=f32).astype(bf16)
        y = y + SCALING * jnp.dot(q, bdn_ref[0], preferred_element_type=f32)
        y_ref[...] = y


def _expand_body(w_ref, o_ref, *, width):
    rows, owidth = o_ref.shape[1], o_ref.shape[2]
    o_ref[0, :, :width] = w_ref[0].astype(jnp.bfloat16)
    o_ref[0, :, width:] = jnp.zeros((rows, owidth - width), jnp.bfloat16)


def _combine_body(z0_ref, z1_ref, w0_ref, w1_ref, o_ref):
    o_ref[...] = w0_ref[...] * z0_ref[...] + w1_ref[...] * z1_ref[...]


def kernel(hidden_states, routing_weights, gate_up_proj, gate_up_proj_bias,
           down_proj, down_proj_bias, lora_A_gate_up, lora_B_gate_up,
           lora_A_down, lora_B_down, router_indices):
    batch, seq, hd = hidden_states.shape
    num_experts, _, two_i = gate_up_proj.shape
    inner = two_i // 2
    rank = lora_A_gate_up.shape[-1]
    tokens = batch * seq
    topk = router_indices.shape[1]
    pairs = tokens * topk
    ntiles = pairs // TM + num_experts
    cap = ntiles * TM

    x = hidden_states.reshape(tokens, hd)
    f32 = jnp.float32
    bf16 = jnp.bfloat16

    e_pair = router_indices.reshape(-1).astype(jnp.int32)
    order = jnp.argsort(e_pair, stable=True).astype(jnp.int32)
    sorted_e = e_pair[order]
    eids = jnp.arange(num_experts, dtype=jnp.int32)
    n_e = jnp.sum(e_pair[None, :] == eids[:, None], axis=1).astype(jnp.int32)
    start_e = jnp.concatenate([jnp.zeros((1,), jnp.int32), jnp.cumsum(n_e)[:-1]])
    ntiles_e = (n_e + TM - 1) // TM
    cumtiles = jnp.cumsum(ntiles_e)
    padded_start_e = TM * jnp.concatenate(
        [jnp.zeros((1,), jnp.int32), cumtiles[:-1]])
    rank_in_group = jnp.arange(pairs, dtype=jnp.int32) - start_e[sorted_e]
    dst = padded_start_e[sorted_e] + rank_in_group
    row_token = jnp.zeros((cap,), jnp.int32).at[dst].set(order // topk)
    pair_pos = jnp.zeros((pairs,), jnp.int32).at[
        (order % topk) * tokens + order // topk].set(dst)
    tile_expert = jnp.searchsorted(
        cumtiles, jnp.arange(ntiles, dtype=jnp.int32), side="right"
    ).astype(jnp.int32)
    w_pair = routing_weights[
        jnp.arange(pairs, dtype=jnp.int32) // topk, e_pair].reshape(tokens, topk)

    wgu = gate_up_proj.astype(bf16)
    bgu_b = gate_up_proj_bias[:, None, :]
    agu = lora_A_gate_up.astype(bf16)
    bgu_l = lora_B_gate_up.astype(bf16)
    rt = 480 if inner % 480 == 0 else inner
    wd = pl.pallas_call(
        functools.partial(_expand_body, width=hd),
        grid=(num_experts, inner // rt),
        in_specs=[pl.BlockSpec((1, rt, hd), lambda e, r: (e, r, 0))],
        out_specs=pl.BlockSpec((1, rt, 2 * hd), lambda e, r: (e, r, 0)),
        out_shape=jax.ShapeDtypeStruct((num_experts, inner, 2 * hd), bf16),
    )(down_proj).reshape(num_experts, two_i, hd)
    bd = down_proj_bias[:, None, :]
    ad = pl.pallas_call(
        functools.partial(_expand_body, width=rank),
        grid=(num_experts, 1),
        in_specs=[pl.BlockSpec((1, inner, rank), lambda e, r: (e, r, 0))],
        out_specs=pl.BlockSpec((1, inner, 2 * rank), lambda e, r: (e, r, 0)),
        out_shape=jax.ShapeDtypeStruct((num_experts, inner, 2 * rank), bf16),
    )(lora_A_down).reshape(num_experts, two_i, rank)
    bdn = lora_B_down.astype(bf16)

    x_pad = _sc_gather_rows(x, row_token)

    def emap(e3):
        return lambda i, te: (jnp.minimum(te[i], num_experts - 1),) + (0,) * e3

    grid_spec = pltpu.PrefetchScalarGridSpec(
        num_scalar_prefetch=1,
        grid=(ntiles,),
        in_specs=[
            pl.BlockSpec((TM, hd), lambda i, te: (i, 0)),
            pl.BlockSpec((1, hd, two_i), emap(2)),
            pl.BlockSpec((1, 1, two_i), emap(2)),
            pl.BlockSpec((1, hd, rank), emap(2)),
            pl.BlockSpec((1, rank, two_i), emap(2)),
            pl.BlockSpec((1, two_i, hd), emap(2)),
            pl.BlockSpec((1, 1, hd), emap(2)),
            pl.BlockSpec((1, two_i, rank), emap(2)),
            pl.BlockSpec((1, rank, hd), emap(2)),
        ],
        out_specs=pl.BlockSpec((TM, hd), lambda i, te: (i, 0)),
    )
    y_pad = pl.pallas_call(
        functools.partial(_grouped_body, num_experts=num_experts),
        grid_spec=grid_spec,
        out_shape=jax.ShapeDtypeStruct((cap, hd), f32),
    )(tile_expert, x_pad, wgu, bgu_b, agu, bgu_l, wd, bd, ad, bdn)

    z = _sc_gather_rows(y_pad, pair_pos)

    tm2 = min(512, tokens)
    nt2 = tokens // tm2
    out = pl.pallas_call(
        _combine_body,
        grid=(nt2,),
        in_specs=[
            pl.BlockSpec((tm2, hd), lambda i: (i, 0)),
            pl.BlockSpec((tm2, hd), lambda i: (i + nt2, 0)),
            pl.BlockSpec((tm2, 1), lambda i: (i, 0)),
            pl.BlockSpec((tm2, 1), lambda i: (i, 0)),
        ],
        out_specs=pl.BlockSpec((tm2, hd), lambda i: (i, 0)),
        out_shape=jax.ShapeDtypeStruct((tokens, hd), f32),
    )(z, z, w_pair[:, 0:1], w_pair[:, 1:2])

    return out.reshape(batch, seq, hd)


# speedup vs baseline: 30.0472x; 1.1383x over previous
import functools

import jax
import jax.numpy as jnp
from jax import lax
from jax.experimental import pallas as pl
from jax.experimental.pallas import tpu as pltpu
from jax.experimental.pallas import tpu_sc as plsc

SCALING = 32.0 / 16.0
ALPHA = 1.702
LIMIT = 7.0

TM = 128
SC_CHUNK = 64


def _sc_gather_rows(table, idx):
    n_rows = idx.shape[0]
    width = table.shape[1]
    info = plsc.get_sparse_core_info()
    nc, ns = info.num_cores, info.num_subcores
    nw = nc * ns
    rows_per_w = n_rows // nw
    assert rows_per_w * nw == n_rows
    chunk = next(c for c in (64, 48, 32, 16, 8) if rows_per_w % c == 0)

    mesh = plsc.VectorSubcoreMesh(core_axis_name="c", subcore_axis_name="s")

    @functools.partial(
        pl.kernel, mesh=mesh,
        out_type=jax.ShapeDtypeStruct((n_rows, width), jnp.float32),
        scratch_types=[
            pltpu.VMEM((chunk,), jnp.int32),
            pltpu.VMEM((chunk, width), jnp.float32),
            pltpu.SemaphoreType.DMA,
        ],
    )
    def gather_k(idx_hbm, table_hbm, out_hbm, idx_v, rows_v, sem):
        wid = lax.axis_index("s") * nc + lax.axis_index("c")
        base = wid * rows_per_w
        for c in range(rows_per_w // chunk):
            off = base + c * chunk
            pltpu.sync_copy(idx_hbm.at[pl.ds(off, chunk)], idx_v)
            pltpu.async_copy(table_hbm.at[idx_v], rows_v, sem).wait()
            pltpu.sync_copy(rows_v, out_hbm.at[pl.ds(off, chunk)])

    return gather_k(idx, table)


def _grouped_body(te_ref, x_ref, wgu_ref, bgu_b_ref, agu_ref, bgu_l_ref,
                  wd_ref, bd_ref, ad_ref, bdn_ref, y_ref, gu_ref,
                  *, num_experts):
    i = pl.program_id(0)
    f32 = jnp.float32
    bf16 = jnp.bfloat16
    inner = wd_ref.shape[1]

    @pl.when(te_ref[i] < num_experts)
    def _():
        x = x_ref[...].astype(bf16)
        p = jnp.dot(x, agu_ref[0], preferred_element_type=f32).astype(bf16)
        gu_t = lax.dot_general(wgu_ref[0], x, (((0,), (1,)), ((), ())),
                               preferred_element_type=f32)
        dlt = lax.dot_general(bgu_l_ref[0], p, (((0,), (1,)), ((), ())),
                              preferred_element_type=f32)
        gu_ref[...] = gu_t + bgu_b_ref[0] + SCALING * dlt
        g = gu_ref[pl.Slice(0, inner, 2), :]
        u = gu_ref[pl.Slice(1, inner, 2), :]
        g = jnp.minimum(g, LIMIT)
        u = jnp.clip(u, -LIMIT, LIMIT)
        glu = g * jax.nn.sigmoid(g * ALPHA)
        gated = ((u + 1.0) * glu).astype(bf16)
        y = lax.dot_general(gated, wd_ref[0], (((0,), (0,)), ((), ())),
                            preferred_element_type=f32) + bd_ref[0]
        q = lax.dot_general(gated, ad_ref[0], (((0,), (0,)), ((), ())),
                            preferred_element_type=f32).astype(bf16)
        y = y + SCALING * jnp.dot(q, bdn_ref[0], preferred_element_type=f32)
        y_ref[...] = y


def _combine_body(z0_ref, z1_ref, w0_ref, w1_ref, o_ref):
    o_ref[...] = w0_ref[...] * z0_ref[...] + w1_ref[...] * z1_ref[...]


def kernel(hidden_states, routing_weights, gate_up_proj, gate_up_proj_bias,
           down_proj, down_proj_bias, lora_A_gate_up, lora_B_gate_up,
           lora_A_down, lora_B_down, router_indices):
    batch, seq, hd = hidden_states.shape
    num_experts, _, two_i = gate_up_proj.shape
    inner = two_i // 2
    rank = lora_A_gate_up.shape[-1]
    tokens = batch * seq
    topk = router_indices.shape[1]
    pairs = tokens * topk
    ntiles = pairs // TM + num_experts
    cap = ntiles * TM

    x = hidden_states.reshape(tokens, hd)
    f32 = jnp.float32
    bf16 = jnp.bfloat16

    e_pair = router_indices.reshape(-1).astype(jnp.int32)
    order = jnp.argsort(e_pair, stable=True).astype(jnp.int32)
    sorted_e = e_pair[order]
    eids = jnp.arange(num_experts, dtype=jnp.int32)
    n_e = jnp.sum(e_pair[None, :] == eids[:, None], axis=1).astype(jnp.int32)
    start_e = jnp.concatenate([jnp.zeros((1,), jnp.int32), jnp.cumsum(n_e)[:-1]])
    ntiles_e = (n_e + TM - 1) // TM
    cumtiles = jnp.cumsum(ntiles_e)
    padded_start_e = TM * jnp.concatenate(
        [jnp.zeros((1,), jnp.int32), cumtiles[:-1]])
    rank_in_group = jnp.arange(pairs, dtype=jnp.int32) - start_e[sorted_e]
    dst = padded_start_e[sorted_e] + rank_in_group
    row_token = jnp.zeros((cap,), jnp.int32).at[dst].set(order // topk)
    pair_pos = jnp.zeros((pairs,), jnp.int32).at[
        (order % topk) * tokens + order // topk].set(dst)
    tile_expert = jnp.searchsorted(
        cumtiles, jnp.arange(ntiles, dtype=jnp.int32), side="right"
    ).astype(jnp.int32)
    w_pair = routing_weights[
        jnp.arange(pairs, dtype=jnp.int32) // topk, e_pair].reshape(tokens, topk)

    wgu = gate_up_proj.astype(bf16)
    bgu_b = gate_up_proj_bias[:, :, None]
    agu = lora_A_gate_up.astype(bf16)
    bgu_l = lora_B_gate_up.astype(bf16)
    wd = down_proj.astype(bf16)
    bd = down_proj_bias[:, None, :]
    ad = lora_A_down.astype(bf16)
    bdn = lora_B_down.astype(bf16)

    x_pad = _sc_gather_rows(x, row_token)

    def emap(e3):
        return lambda i, te: (jnp.minimum(te[i], num_experts - 1),) + (0,) * e3

    grid_spec = pltpu.PrefetchScalarGridSpec(
        num_scalar_prefetch=1,
        grid=(ntiles,),
        in_specs=[
            pl.BlockSpec((TM, hd), lambda i, te: (i, 0)),
            pl.BlockSpec((1, hd, two_i), emap(2)),
            pl.BlockSpec((1, two_i, 1), emap(2)),
            pl.BlockSpec((1, hd, rank), emap(2)),
            pl.BlockSpec((1, rank, two_i), emap(2)),
            pl.BlockSpec((1, inner, hd), emap(2)),
            pl.BlockSpec((1, 1, hd), emap(2)),
            pl.BlockSpec((1, inner, rank), emap(2)),
            pl.BlockSpec((1, rank, hd), emap(2)),
        ],
        out_specs=pl.BlockSpec((TM, hd), lambda i, te: (i, 0)),
        scratch_shapes=[pltpu.VMEM((two_i, TM), jnp.float32)],
    )
    y_pad = pl.pallas_call(
        functools.partial(_grouped_body, num_experts=num_experts),
        grid_spec=grid_spec,
        out_shape=jax.ShapeDtypeStruct((cap, hd), f32),
    )(tile_expert, x_pad, wgu, bgu_b, agu, bgu_l, wd, bd, ad, bdn)

    z = _sc_gather_rows(y_pad, pair_pos)

    tm2 = min(512, tokens)
    nt2 = tokens // tm2
    out = pl.pallas_call(
        _combine_body,
        grid=(nt2,),
        in_specs=[
            pl.BlockSpec((tm2, hd), lambda i: (i, 0)),
            pl.BlockSpec((tm2, hd), lambda i: (i + nt2, 0)),
            pl.BlockSpec((tm2, 1), lambda i: (i, 0)),
            pl.BlockSpec((tm2, 1), lambda i: (i, 0)),
        ],
        out_specs=pl.BlockSpec((tm2, hd), lambda i: (i, 0)),
        out_shape=jax.ShapeDtypeStruct((tokens, hd), f32),
    )(z, z, w_pair[:, 0:1], w_pair[:, 1:2])

    return out.reshape(batch, seq, hd)


# speedup vs baseline: 31.2757x; 1.0409x over previous
import functools

import jax
import jax.numpy as jnp
from jax import lax
from jax.experimental import pallas as pl
from jax.experimental.pallas import tpu as pltpu
from jax.experimental.pallas import tpu_sc as plsc

SCALING = 32.0 / 16.0
ALPHA = 1.702
LIMIT = 7.0

TM = 256
SC_CHUNK = 64


def _sc_gather_rows(table, idx):
    n_rows = idx.shape[0]
    width = table.shape[1]
    info = plsc.get_sparse_core_info()
    nc, ns = info.num_cores, info.num_subcores
    nw = nc * ns
    rows_per_w = n_rows // nw
    assert rows_per_w * nw == n_rows
    chunk = next(c for c in (64, 48, 32, 16, 8) if rows_per_w % c == 0)

    mesh = plsc.VectorSubcoreMesh(core_axis_name="c", subcore_axis_name="s")

    @functools.partial(
        pl.kernel, mesh=mesh,
        out_type=jax.ShapeDtypeStruct((n_rows, width), jnp.float32),
        scratch_types=[
            pltpu.VMEM((chunk,), jnp.int32),
            pltpu.VMEM((chunk, width), jnp.float32),
            pltpu.SemaphoreType.DMA,
        ],
    )
    def gather_k(idx_hbm, table_hbm, out_hbm, idx_v, rows_v, sem):
        wid = lax.axis_index("s") * nc + lax.axis_index("c")
        base = wid * rows_per_w
        for c in range(rows_per_w // chunk):
            off = base + c * chunk
            pltpu.sync_copy(idx_hbm.at[pl.ds(off, chunk)], idx_v)
            pltpu.async_copy(table_hbm.at[idx_v], rows_v, sem).wait()
            pltpu.sync_copy(rows_v, out_hbm.at[pl.ds(off, chunk)])

    return gather_k(idx, table)


def _gateup_body(te_ref, x_ref, wgu_ref, bgu_b_ref, agu_ref, bgu_l_ref,
                 gated_ref, gu_ref, *, num_experts):
    i = pl.program_id(0)
    f32 = jnp.float32
    bf16 = jnp.bfloat16
    inner = gated_ref.shape[0]

    @pl.when(te_ref[i] < num_experts)
    def _():
        x = x_ref[...]
        p = jnp.dot(x, agu_ref[0], preferred_element_type=f32).astype(bf16)
        gu_t = lax.dot_general(wgu_ref[0], x, (((0,), (1,)), ((), ())),
                               preferred_element_type=f32)
        dlt = lax.dot_general(bgu_l_ref[0], p, (((0,), (1,)), ((), ())),
                              preferred_element_type=f32)
        gu_t = gu_t + bgu_b_ref[0] + SCALING * dlt
        for k in range(gu_t.shape[1] // 128):
            gu_ref[...] = gu_t[:, k * 128:(k + 1) * 128]
            g = jnp.minimum(gu_ref[pl.Slice(0, inner, 2), :], LIMIT)
            u = jnp.clip(gu_ref[pl.Slice(1, inner, 2), :], -LIMIT, LIMIT)
            glu = g * jax.nn.sigmoid(g * ALPHA)
            gated_ref[:, k * 128:(k + 1) * 128] = ((u + 1.0) * glu).astype(bf16)


def _down_body(te_ref, gated_ref, wd_ref, bd_ref, ad_ref, bdn_ref, y_ref,
               *, num_experts):
    i = pl.program_id(0)
    f32 = jnp.float32
    bf16 = jnp.bfloat16

    @pl.when(te_ref[i] < num_experts)
    def _():
        gated = gated_ref[...]
        y = lax.dot_general(gated, wd_ref[0], (((0,), (0,)), ((), ())),
                            preferred_element_type=f32) + bd_ref[0]
        q = lax.dot_general(gated, ad_ref[0], (((0,), (0,)), ((), ())),
                            preferred_element_type=f32).astype(bf16)
        y = y + SCALING * jnp.dot(q, bdn_ref[0], preferred_element_type=f32)
        y_ref[...] = y


def _combine_body(z0_ref, z1_ref, w0_ref, w1_ref, o_ref):
    o_ref[...] = w0_ref[...] * z0_ref[...] + w1_ref[...] * z1_ref[...]


def kernel(hidden_states, routing_weights, gate_up_proj, gate_up_proj_bias,
           down_proj, down_proj_bias, lora_A_gate_up, lora_B_gate_up,
           lora_A_down, lora_B_down, router_indices):
    batch, seq, hd = hidden_states.shape
    num_experts, _, two_i = gate_up_proj.shape
    inner = two_i // 2
    rank = lora_A_gate_up.shape[-1]
    tokens = batch * seq
    topk = router_indices.shape[1]
    pairs = tokens * topk
    ntiles = pairs // TM + num_experts
    cap = ntiles * TM

    x = hidden_states.reshape(tokens, hd)
    f32 = jnp.float32
    bf16 = jnp.bfloat16

    e_pair = router_indices.reshape(-1).astype(jnp.int32)
    order = jnp.argsort(e_pair, stable=True).astype(jnp.int32)
    sorted_e = e_pair[order]
    eids = jnp.arange(num_experts, dtype=jnp.int32)
    n_e = jnp.sum(e_pair[None, :] == eids[:, None], axis=1).astype(jnp.int32)
    start_e = jnp.concatenate([jnp.zeros((1,), jnp.int32), jnp.cumsum(n_e)[:-1]])
    ntiles_e = (n_e + TM - 1) // TM
    cumtiles = jnp.cumsum(ntiles_e)
    padded_start_e = TM * jnp.concatenate(
        [jnp.zeros((1,), jnp.int32), cumtiles[:-1]])
    rank_in_group = jnp.arange(pairs, dtype=jnp.int32) - start_e[sorted_e]
    dst = padded_start_e[sorted_e] + rank_in_group
    row_token = jnp.zeros((cap,), jnp.int32).at[dst].set(order // topk)
    pair_pos = jnp.zeros((pairs,), jnp.int32).at[
        (order % topk) * tokens + order // topk].set(dst)
    tile_expert = jnp.searchsorted(
        cumtiles, jnp.arange(ntiles, dtype=jnp.int32), side="right"
    ).astype(jnp.int32)
    w_pair = routing_weights[
        jnp.arange(pairs, dtype=jnp.int32) // topk, e_pair].reshape(tokens, topk)

    wgu = gate_up_proj.astype(bf16)
    bgu_b = gate_up_proj_bias[:, :, None]
    agu = lora_A_gate_up.astype(bf16)
    bgu_l = lora_B_gate_up.astype(bf16)
    wd = down_proj.astype(bf16)
    bd = down_proj_bias[:, None, :]
    ad = lora_A_down.astype(bf16)
    bdn = lora_B_down.astype(bf16)

    x_pad = _sc_gather_rows(x, row_token).astype(bf16)

    def emap(e3):
        return lambda i, te: (jnp.minimum(te[i], num_experts - 1),) + (0,) * e3

    gu_grid = pltpu.PrefetchScalarGridSpec(
        num_scalar_prefetch=1,
        grid=(ntiles,),
        in_specs=[
            pl.BlockSpec((TM, hd), lambda i, te: (i, 0)),
            pl.BlockSpec((1, hd, two_i), emap(2)),
            pl.BlockSpec((1, two_i, 1), emap(2)),
            pl.BlockSpec((1, hd, rank), emap(2)),
            pl.BlockSpec((1, rank, two_i), emap(2)),
        ],
        out_specs=pl.BlockSpec((inner, TM), lambda i, te: (0, i)),
        scratch_shapes=[pltpu.VMEM((two_i, 128), jnp.float32)],
    )
    gated = pl.pallas_call(
        functools.partial(_gateup_body, num_experts=num_experts),
        grid_spec=gu_grid,
        out_shape=jax.ShapeDtypeStruct((inner, cap), bf16),
    )(tile_expert, x_pad, wgu, bgu_b, agu, bgu_l)

    dn_grid = pltpu.PrefetchScalarGridSpec(
        num_scalar_prefetch=1,
        grid=(ntiles,),
        in_specs=[
            pl.BlockSpec((inner, TM), lambda i, te: (0, i)),
            pl.BlockSpec((1, inner, hd), emap(2)),
            pl.BlockSpec((1, 1, hd), emap(2)),
            pl.BlockSpec((1, inner, rank), emap(2)),
            pl.BlockSpec((1, rank, hd), emap(2)),
        ],
        out_specs=pl.BlockSpec((TM, hd), lambda i, te: (i, 0)),
    )
    y_pad = pl.pallas_call(
        functools.partial(_down_body, num_experts=num_experts),
        grid_spec=dn_grid,
        out_shape=jax.ShapeDtypeStruct((cap, hd), f32),
    )(tile_expert, gated, wd, bd, ad, bdn)

    z = _sc_gather_rows(y_pad, pair_pos)

    tm2 = min(512, tokens)
    nt2 = tokens // tm2
    out = pl.pallas_call(
        _combine_body,
        grid=(nt2,),
        in_specs=[
            pl.BlockSpec((tm2, hd), lambda i: (i, 0)),
            pl.BlockSpec((tm2, hd), lambda i: (i + nt2, 0)),
            pl.BlockSpec((tm2, 1), lambda i: (i, 0)),
            pl.BlockSpec((tm2, 1), lambda i: (i, 0)),
        ],
        out_specs=pl.BlockSpec((tm2, hd), lambda i: (i, 0)),
        out_shape=jax.ShapeDtypeStruct((tokens, hd), f32),
    )(z, z, w_pair[:, 0:1], w_pair[:, 1:2])

    return out.reshape(batch, seq, hd)
